# baseline (speedup 1.0000x reference)
import functools

import jax
import jax.numpy as jnp
from jax import lax
from jax.experimental import pallas as pl
from jax.experimental.pallas import tpu as pltpu
from jax.experimental.pallas import tpu_sc as plsc

N_COMP = 64
ROWS = 16384
Z_COLS = 128
NC, NS, LANES = 2, 16, 16
NW = NC * NS
ROWS_W = ROWS // NW
CHUNK = 128
GROUPS = CHUNK // LANES

_DN = lax.GatherDimensionNumbers(
    offset_dims=(), collapsed_slice_dims=(0,), start_index_map=(0,))


def _rot(x, idx):
    return lax.gather(x, idx.reshape(LANES, 1), _DN, slice_sizes=(1,),
                      mode=lax.GatherScatterMode.PROMISE_IN_BOUNDS)


@functools.partial(
    pl.kernel,
    out_type=(
        jax.ShapeDtypeStruct((ROWS, Z_COLS), jnp.float32),
        jax.ShapeDtypeStruct((NW, N_COMP), jnp.float32),
        jax.ShapeDtypeStruct((NW, LANES), jnp.float32),
    ),
    mesh=plsc.VectorSubcoreMesh(
        core_axis_name="c", subcore_axis_name="s",
        num_cores=NC, num_subcores=NS,
    ),
    scratch_types=(
        pltpu.VMEM((CHUNK, N_COMP), jnp.float32),
        pltpu.VMEM((CHUNK, N_COMP), jnp.float32),
        pltpu.VMEM((N_COMP,), jnp.float32),
        pltpu.VMEM((LANES,), jnp.float32),
        pltpu.SemaphoreType.DMA,
        pltpu.SemaphoreType.DMA,
        pltpu.SemaphoreType.DMA,
    ),
    compiler_params=pltpu.CompilerParams(
        needs_layout_passes=False, use_tc_tiling_on_sc=True),
)
def _sc_pass(z_hbm, zc_hbm, zout_hbm, pop_hbm, conf_hbm,
             buf_a, buf_b, pop, conf, sem_a, sem_b, sem_z):
    wid = lax.axis_index("s") * NC + lax.axis_index("c")
    r0 = wid * ROWS_W

    zcopy = pltpu.async_copy(
        z_hbm.at[pl.ds(r0, ROWS_W), :], zout_hbm.at[pl.ds(r0, ROWS_W), :],
        sem_z)

    bufs = (buf_a, buf_b)
    sems = (sem_a, sem_b)

    def start(i):
        return pltpu.async_copy(
            zc_hbm.at[pl.ds(r0 + i * CHUNK, CHUNK), :], bufs[i % 2],
            sems[i % 2])

    cps = [start(0), start(1)]

    zeros16 = jnp.zeros((LANES,), jnp.float32)
    for k in range(N_COMP // LANES):
        pop[pl.ds(k * LANES, LANES)] = zeros16

    lanes16 = lax.iota(jnp.int32, LANES)
    ones16 = jnp.ones((LANES,), jnp.float32)
    mask0 = lanes16 == 0
    kvecs = [jnp.full((LANES,), k, jnp.int32) for k in range(4)]
    rots = [(lanes16 + s) & (LANES - 1) for s in (8, 4, 2, 1)]

    def make_body(buf):
        def row_calc(r, conf_acc):
            v0 = buf[r, pl.ds(0, LANES)]
            v1 = buf[r, pl.ds(LANES, LANES)]
            v2 = buf[r, pl.ds(2 * LANES, LANES)]
            v3 = buf[r, pl.ds(3 * LANES, LANES)]
            g1 = v1 > v0
            m01 = jnp.where(g1, v1, v0)
            k01 = jnp.where(g1, kvecs[1], kvecs[0])
            g2 = v3 > v2
            m23 = jnp.where(g2, v3, v2)
            k23 = jnp.where(g2, kvecs[3], kvecs[2])
            g3 = m23 > m01
            mm = jnp.where(g3, m23, m01)
            kk = jnp.where(g3, k23, k01)
            cand = kk * LANES + lanes16
            for ridx in rots:
                mm2 = _rot(mm, ridx)
                cd2 = _rot(cand, ridx)
                take = (mm2 > mm) | ((mm2 == mm) & (cd2 < cand))
                mm = jnp.where(take, mm2, mm)
                cand = jnp.where(take, cd2, cand)
            plsc.store_scatter(pop, [cand], ones16, mask=mask0)
            return conf_acc + mm

        def g_body(g, conf_acc):
            rb = g * LANES
            for j in range(LANES):
                conf_acc = row_calc(rb + j, conf_acc)
            return conf_acc

        return g_body

    body_a = make_body(buf_a)
    body_b = make_body(buf_b)

    cps[0].wait()
    conf_acc = lax.fori_loop(0, GROUPS, body_a, zeros16)
    cp2 = start(2)
    cps[1].wait()
    conf_acc = lax.fori_loop(0, GROUPS, body_b, conf_acc)
    cp3 = start(3)
    cp2.wait()
    conf_acc = lax.fori_loop(0, GROUPS, body_a, conf_acc)
    cp3.wait()
    conf_acc = lax.fori_loop(0, GROUPS, body_b, conf_acc)

    conf[...] = conf_acc
    pltpu.sync_copy(pop, pop_hbm.at[wid])
    pltpu.sync_copy(conf, conf_hbm.at[wid])
    zcopy.wait()


def _merge_body(pop_ref, conf_ref, np_ref, cm_ref):
    present = jnp.max(pop_ref[...], axis=0, keepdims=True)
    num_pop = jnp.sum(jnp.where(present > 0.0, 1.0, 0.0))
    np_ref[...] = num_pop.reshape(1, 1)
    cm_ref[...] = (jnp.sum(conf_ref[...]) * (1.0 / (ROWS * LANES))).reshape(1, 1)


_merge = pl.pallas_call(
    _merge_body,
    out_shape=(
        jax.ShapeDtypeStruct((1, 1), jnp.float32),
        jax.ShapeDtypeStruct((1, 1), jnp.float32),
    ),
)


def kernel(z, z_cat):
    z_out, pop_part, conf_part = _sc_pass(z, z_cat)
    num_pop, conf_mean = _merge(pop_part, conf_part)
    return (z_out, num_pop[0, 0], conf_mean[0, 0])


# speedup vs baseline: 6.5080x; 6.5080x over previous
import functools

import jax
import jax.numpy as jnp
from jax import lax
from jax.experimental import pallas as pl
from jax.experimental.pallas import tpu as pltpu
from jax.experimental.pallas import tpu_sc as plsc

N_COMP = 64
ROWS = 16384
NC, NS, LANES = 2, 16, 16
NW = NC * NS
ROWS_W = ROWS // NW
GRID = 8
BROWS = 16


def _prep_body(zc_ref, conf_ref, arg_ref):
    x = zc_ref[...]
    m = jnp.max(x, axis=2)
    iot = lax.broadcasted_iota(jnp.int32, (BROWS, 128, N_COMP), 2)
    arg_ref[...] = jnp.min(jnp.where(x == m[:, :, None], iot, N_COMP), axis=2)
    conf_ref[...] = m


_prep = pl.pallas_call(
    _prep_body,
    grid=(GRID,),
    in_specs=[pl.BlockSpec((BROWS, 128, N_COMP), lambda i: (i, 0, 0))],
    out_specs=(
        pl.BlockSpec((BROWS, 128), lambda i: (i, 0)),
        pl.BlockSpec((BROWS, 128), lambda i: (i, 0)),
    ),
    out_shape=(
        jax.ShapeDtypeStruct((128, 128), jnp.float32),
        jax.ShapeDtypeStruct((128, 128), jnp.int32),
    ),
)


@functools.partial(
    pl.kernel,
    out_type=(
        jax.ShapeDtypeStruct((NW, N_COMP), jnp.float32),
        jax.ShapeDtypeStruct((NW, LANES), jnp.float32),
    ),
    mesh=plsc.VectorSubcoreMesh(
        core_axis_name="c", subcore_axis_name="s",
        num_cores=NC, num_subcores=NS,
    ),
    scratch_types=(
        pltpu.VMEM((ROWS_W // 128, 128), jnp.float32),
        pltpu.VMEM((ROWS_W // 128, 128), jnp.int32),
        pltpu.VMEM((N_COMP,), jnp.float32),
        pltpu.VMEM((LANES,), jnp.float32),
        pltpu.SemaphoreType.DMA,
        pltpu.SemaphoreType.DMA,
    ),
    compiler_params=pltpu.CompilerParams(needs_layout_passes=False),
)
def _sc_hist(conf_hbm, arg_hbm, pop_hbm, confp_hbm,
             cbuf, abuf, pop, confv, sem_c, sem_a):
    wid = lax.axis_index("s") * NC + lax.axis_index("c")
    r4 = wid * (ROWS_W // 128)
    cpa = pltpu.async_copy(
        arg_hbm.at[pl.ds(r4, ROWS_W // 128), :], abuf, sem_a)
    cpc = pltpu.async_copy(
        conf_hbm.at[pl.ds(r4, ROWS_W // 128), :], cbuf, sem_c)

    zeros16 = jnp.zeros((LANES,), jnp.float32)
    for k in range(N_COMP // LANES):
        pop[pl.ds(k * LANES, LANES)] = zeros16
    ones16 = jnp.ones((LANES,), jnp.float32)

    cpa.wait()
    for t in range(ROWS_W // LANES):
        idx = abuf[t // 8, pl.ds((t % 8) * LANES, LANES)]
        plsc.store_scatter(pop, [idx], ones16)
    cpc.wait()
    acc = zeros16
    for t in range(ROWS_W // LANES):
        acc = acc + cbuf[t // 8, pl.ds((t % 8) * LANES, LANES)]
    confv[...] = acc
    pltpu.sync_copy(pop, pop_hbm.at[wid])
    pltpu.sync_copy(confv, confp_hbm.at[wid])


def _merge_body(pop_ref, conf_ref, np_ref, cm_ref):
    present = jnp.max(pop_ref[...], axis=0, keepdims=True)
    num_pop = jnp.sum(jnp.where(present > 0.0, 1.0, 0.0))
    np_ref[...] = num_pop.reshape(1, 1)
    cm_ref[...] = (jnp.sum(conf_ref[...]) * (1.0 / ROWS)).reshape(1, 1)


_merge = pl.pallas_call(
    _merge_body,
    out_shape=(
        jax.ShapeDtypeStruct((1, 1), jnp.float32),
        jax.ShapeDtypeStruct((1, 1), jnp.float32),
    ),
)


def kernel(z, z_cat):
    zc3 = z_cat.reshape(128, 128, N_COMP)
    conf2, arg2 = _prep(zc3)
    pop_part, conf_part = _sc_hist(conf2, arg2)
    num_pop, conf_mean = _merge(pop_part, conf_part)
    return (z, num_pop[0, 0], conf_mean[0, 0])


# speedup vs baseline: 6.5308x; 1.0035x over previous
import functools

import jax
import jax.numpy as jnp
from jax import lax
from jax.experimental import pallas as pl
from jax.experimental.pallas import tpu as pltpu
from jax.experimental.pallas import tpu_sc as plsc

N_COMP = 64
ROWS = 16384
NC, NS, LANES = 2, 16, 16
NW = NC * NS
ROWS_W = ROWS // NW
GRID = 8
BROWS = 16


BLK = ROWS // GRID


def _prep_body(zc_ref, conf_ref, arg_ref):
    x = zc_ref[...]
    m = jnp.max(x, axis=1, keepdims=True)
    iot = lax.broadcasted_iota(jnp.int32, (BLK, N_COMP), 1)
    a = jnp.min(jnp.where(x == m, iot, N_COMP), axis=1)
    conf_ref[...] = m.reshape(BLK // 128, 128)
    arg_ref[...] = a.reshape(BLK // 128, 128)


_prep = pl.pallas_call(
    _prep_body,
    grid=(GRID,),
    in_specs=[pl.BlockSpec((BLK, N_COMP), lambda i: (i, 0))],
    out_specs=(
        pl.BlockSpec((BLK // 128, 128), lambda i: (i, 0)),
        pl.BlockSpec((BLK // 128, 128), lambda i: (i, 0)),
    ),
    out_shape=(
        jax.ShapeDtypeStruct((128, 128), jnp.float32),
        jax.ShapeDtypeStruct((128, 128), jnp.int32),
    ),
)


@functools.partial(
    pl.kernel,
    out_type=(
        jax.ShapeDtypeStruct((NW, N_COMP), jnp.float32),
        jax.ShapeDtypeStruct((NW, LANES), jnp.float32),
    ),
    mesh=plsc.VectorSubcoreMesh(
        core_axis_name="c", subcore_axis_name="s",
        num_cores=NC, num_subcores=NS,
    ),
    scratch_types=(
        pltpu.VMEM((ROWS_W // 128, 128), jnp.float32),
        pltpu.VMEM((ROWS_W // 128, 128), jnp.int32),
        pltpu.VMEM((N_COMP,), jnp.float32),
        pltpu.VMEM((LANES,), jnp.float32),
        pltpu.SemaphoreType.DMA,
        pltpu.SemaphoreType.DMA,
    ),
    compiler_params=pltpu.CompilerParams(needs_layout_passes=False),
)
def _sc_hist(conf_hbm, arg_hbm, pop_hbm, confp_hbm,
             cbuf, abuf, pop, confv, sem_c, sem_a):
    wid = lax.axis_index("s") * NC + lax.axis_index("c")
    r4 = wid * (ROWS_W // 128)
    cpa = pltpu.async_copy(
        arg_hbm.at[pl.ds(r4, ROWS_W // 128), :], abuf, sem_a)
    cpc = pltpu.async_copy(
        conf_hbm.at[pl.ds(r4, ROWS_W // 128), :], cbuf, sem_c)

    zeros16 = jnp.zeros((LANES,), jnp.float32)
    for k in range(N_COMP // LANES):
        pop[pl.ds(k * LANES, LANES)] = zeros16
    ones16 = jnp.ones((LANES,), jnp.float32)

    cpa.wait()
    for t in range(ROWS_W // LANES):
        idx = abuf[t // 8, pl.ds((t % 8) * LANES, LANES)]
        plsc.store_scatter(pop, [idx], ones16)
    cpc.wait()
    acc = zeros16
    for t in range(ROWS_W // LANES):
        acc = acc + cbuf[t // 8, pl.ds((t % 8) * LANES, LANES)]
    confv[...] = acc
    pltpu.sync_copy(pop, pop_hbm.at[wid])
    pltpu.sync_copy(confv, confp_hbm.at[wid])


def _merge_body(pop_ref, conf_ref, np_ref, cm_ref):
    present = jnp.max(pop_ref[...], axis=0, keepdims=True)
    num_pop = jnp.sum(jnp.where(present > 0.0, 1.0, 0.0))
    np_ref[...] = num_pop.reshape(1, 1)
    cm_ref[...] = (jnp.sum(conf_ref[...]) * (1.0 / ROWS)).reshape(1, 1)


_merge = pl.pallas_call(
    _merge_body,
    out_shape=(
        jax.ShapeDtypeStruct((1, 1), jnp.float32),
        jax.ShapeDtypeStruct((1, 1), jnp.float32),
    ),
)


def kernel(z, z_cat):
    conf2, arg2 = _prep(z_cat)
    pop_part, conf_part = _sc_hist(conf2, arg2)
    num_pop, conf_mean = _merge(pop_part, conf_part)
    return (z, num_pop[0, 0], conf_mean[0, 0])
